# Initial kernel scaffold; baseline (speedup 1.0000x reference)
import jax
import jax.numpy as jnp
from jax.experimental import pallas as pl


def kernel(decoder_attention, decoder_last_hidden_state, enc_input_ids, logits, decoder_input_embeds, encoder_last_hidden_state, w_logits, b_logits, w_dec, b_dec, w_enc, b_enc, bias):
    raise NotImplementedError("write your pallas kernel here")


# speedup vs baseline: 1.4567x; 1.4567x over previous
import functools

import jax
import jax.numpy as jnp
from jax import lax
from jax.experimental import pallas as pl
from jax.experimental.pallas import tpu as pltpu
from jax.experimental.pallas import tpu_sc as plsc

B, H, T, S, V, D = 4, 12, 256, 1024, 32000, 768

NC, NS, L = 2, 16, 16
W = NC * NS
RW = (B * T) // W
CHUNK = 128
G = (RW * S) // CHUNK
ROWCH = S // CHUNK
INFLIGHT = 8
EPS = 1e-12

def _build_indices(wid, ids_hbm, ids_v, idx_v):
    r0 = wid * RW
    b = r0 // T
    pltpu.sync_copy(ids_hbm.at[b], ids_v)

    def row_body(i, _):
        r_v = (r0 + i) * V

        def chunk_body(j, _):
            for k in range(CHUNK // L):
                idx_v[i * ROWCH + j, pl.ds(k * L, L)] = (
                    ids_v[pl.ds(j * CHUNK + k * L, L)] + r_v)
            return 0

        return lax.fori_loop(0, ROWCH, chunk_body, 0)

    lax.fori_loop(0, RW, row_body, 0)


@functools.cache
def _sc_kernels():
    mesh = plsc.VectorSubcoreMesh(
        core_axis_name="c", subcore_axis_name="s",
        num_cores=NC, num_subcores=NS)
    scratch = [
        pltpu.VMEM((S,), jnp.int32),
        pltpu.VMEM((G, CHUNK), jnp.int32),
        pltpu.VMEM((G, CHUNK), jnp.float32),
        pltpu.SemaphoreType.DMA,
    ]

    @functools.partial(
        pl.kernel, mesh=mesh,
        out_type=jax.ShapeDtypeStruct((W, G, CHUNK), jnp.float32),
        scratch_types=scratch)
    def sc_gather(logits_hbm, ids_hbm, out_hbm, ids_v, idx_v, data_v, sem):
        wid = lax.axis_index("s") * NC + lax.axis_index("c")
        _build_indices(wid, ids_hbm, ids_v, idx_v)

        def fire_drain(gg, _):
            descs = [
                pltpu.async_copy(
                    logits_hbm.at[idx_v.at[gg * INFLIGHT + u]],
                    data_v.at[gg * INFLIGHT + u], sem)
                for u in range(INFLIGHT)
            ]
            for d in descs:
                d.wait()
            return 0

        lax.fori_loop(0, G // INFLIGHT, fire_drain, 0)
        pltpu.sync_copy(data_v, out_hbm.at[wid])

    @functools.partial(pl.kernel, mesh=mesh, out_type=(),
                       scratch_types=scratch)
    def sc_scatter(val_hbm, ids_hbm, out_hbm, ids_v, idx_v, data_v, sem):
        wid = lax.axis_index("s") * NC + lax.axis_index("c")
        _build_indices(wid, ids_hbm, ids_v, idx_v)
        pltpu.sync_copy(val_hbm.at[wid], data_v)

        def fire_drain(gg, _):
            descs = [
                pltpu.async_copy(
                    data_v.at[gg * INFLIGHT + u],
                    out_hbm.at[idx_v.at[gg * INFLIGHT + u]], sem)
                for u in range(INFLIGHT)
            ]
            for d in descs:
                d.wait()
            return 0

        lax.fori_loop(0, G // INFLIGHT, fire_drain, 0)

    return sc_gather, sc_scatter


TT = 32


def _dense_body(logits_ref, datt_ref, enc_ref, dh_ref, de_ref, idsf_ref,
                lgg_ref, w1_ref, w2_ref, we_ref, bsum_ref,
                out_ref, val_ref, m_ref):
    hi = jax.lax.Precision.HIGHEST
    t = pl.program_id(1)
    ids = idsf_ref[0, 0, :]

    @pl.when(t == 0)
    def _():
        m_ref[...] = (ids[:, None] == ids[None, :]).astype(jnp.float32)

    catt = jnp.mean(datt_ref[0], axis=0)
    we = lax.dot(enc_ref[0], we_ref[...], precision=hi)
    z = (lax.dot(dh_ref[0], w1_ref[...], precision=hi)
         + lax.dot(de_ref[0], w2_ref[...], precision=hi)
         + lax.dot(catt, we, precision=hi)
         + bsum_ref[0, 0])
    p = jax.nn.sigmoid(z)

    l = logits_ref[0]
    m = jnp.max(l, axis=-1, keepdims=True)
    e = jnp.exp(l - m)
    den = jnp.sum(e, axis=-1, keepdims=True)
    a = (1.0 - p) / den
    out_ref[0] = jnp.log(a * e + EPS)

    cm = lax.dot(catt, m_ref[...], precision=hi)
    val_ref[0] = jnp.log(a * jnp.exp(lgg_ref[0] - m) + p * cm + EPS)


def _dense_pass(logits, datt, enc, dh, de, idsf, lgg, w1, w2, we, bsum):
    grid = (B, T // TT)
    return pl.pallas_call(
        _dense_body,
        grid=grid,
        in_specs=[
            pl.BlockSpec((1, TT, V), lambda b, t: (b, t, 0)),
            pl.BlockSpec((1, H, TT, S), lambda b, t: (b, 0, t, 0)),
            pl.BlockSpec((1, S, D), lambda b, t: (b, 0, 0)),
            pl.BlockSpec((1, TT, D), lambda b, t: (b, t, 0)),
            pl.BlockSpec((1, TT, D), lambda b, t: (b, t, 0)),
            pl.BlockSpec((1, 1, S), lambda b, t: (b, 0, 0)),
            pl.BlockSpec((1, TT, S), lambda b, t: (b, t, 0)),
            pl.BlockSpec((D, 1), lambda b, t: (0, 0)),
            pl.BlockSpec((D, 1), lambda b, t: (0, 0)),
            pl.BlockSpec((D, 1), lambda b, t: (0, 0)),
            pl.BlockSpec(memory_space=pltpu.SMEM),
        ],
        out_specs=[
            pl.BlockSpec((1, TT, V), lambda b, t: (b, t, 0)),
            pl.BlockSpec((1, TT, S), lambda b, t: (b, t, 0)),
        ],
        out_shape=[
            jax.ShapeDtypeStruct((B, T, V), jnp.float32),
            jax.ShapeDtypeStruct((B, T, S), jnp.float32),
        ],
        scratch_shapes=[pltpu.VMEM((S, S), jnp.float32)],
    )(logits, datt, enc, dh, de, idsf, lgg, w1, w2, we, bsum)


def kernel(decoder_attention, decoder_last_hidden_state, enc_input_ids, logits,
           decoder_input_embeds, encoder_last_hidden_state,
           w_logits, b_logits, w_dec, b_dec, w_enc, b_enc, bias):
    ids32 = enc_input_ids.astype(jnp.int32)
    idsf = ids32.astype(jnp.float32).reshape(B, 1, S)
    bsum = (b_logits + b_dec + b_enc + bias).reshape(1, 1)

    sc_gather, sc_scatter = _sc_kernels()
    lg3 = sc_gather(logits.reshape(-1), ids32)
    lgg = lg3.reshape(B, T, S)

    out_pre, val = _dense_pass(
        logits, decoder_attention, encoder_last_hidden_state,
        decoder_last_hidden_state, decoder_input_embeds, idsf, lgg,
        w_logits, w_dec, w_enc, bsum)

    out_ref = jax.new_ref(out_pre.reshape(-1))
    sc_scatter(val.reshape(W, G, CHUNK), ids32, out_ref)
    return out_ref[...].reshape(B, T, V)


# speedup vs baseline: 1.6426x; 1.1276x over previous
import functools

import jax
import jax.numpy as jnp
from jax import lax
from jax.experimental import pallas as pl
from jax.experimental.pallas import tpu as pltpu
from jax.experimental.pallas import tpu_sc as plsc

B, H, T, S, V, D = 4, 12, 256, 1024, 32000, 768

NC, NS, L = 2, 16, 16
W = NC * NS
RW = (B * T) // W
CHUNK = 128
G = (RW * S) // CHUNK
ROWCH = S // CHUNK
INFLIGHT = 8
EPS = 1e-12

def _build_indices(wid, ids_hbm, ids_v, idx_v):
    r0 = wid * RW
    b = r0 // T
    pltpu.sync_copy(ids_hbm.at[b], ids_v)

    def row_body(i, _):
        r_v = (r0 + i) * V

        def chunk_body(j, _):
            for k in range(CHUNK // L):
                idx_v[pl.ds(i * S + j * CHUNK + k * L, L)] = (
                    ids_v[pl.ds(j * CHUNK + k * L, L)] + r_v)
            return 0

        return lax.fori_loop(0, ROWCH, chunk_body, 0)

    lax.fori_loop(0, RW, row_body, 0)


@functools.cache
def _sc_kernels():
    mesh = plsc.VectorSubcoreMesh(
        core_axis_name="c", subcore_axis_name="s",
        num_cores=NC, num_subcores=NS)
    scratch = [
        pltpu.VMEM((S,), jnp.int32),
        pltpu.VMEM((RW * S,), jnp.int32),
        pltpu.VMEM((RW * S,), jnp.float32),
        pltpu.SemaphoreType.DMA,
    ]

    @functools.partial(
        pl.kernel, mesh=mesh,
        out_type=jax.ShapeDtypeStruct((W, RW * S), jnp.float32),
        scratch_types=scratch)
    def sc_gather(logits_hbm, ids_hbm, out_hbm, ids_v, idx_v, data_v, sem):
        wid = lax.axis_index("s") * NC + lax.axis_index("c")
        _build_indices(wid, ids_hbm, ids_v, idx_v)
        pltpu.async_copy(logits_hbm.at[idx_v], data_v, sem).wait()
        pltpu.sync_copy(data_v, out_hbm.at[wid])

    @functools.partial(pl.kernel, mesh=mesh, out_type=(),
                       scratch_types=scratch)
    def sc_scatter(val_hbm, ids_hbm, out_hbm, ids_v, idx_v, data_v, sem):
        wid = lax.axis_index("s") * NC + lax.axis_index("c")
        _build_indices(wid, ids_hbm, ids_v, idx_v)
        pltpu.sync_copy(val_hbm.at[wid], data_v)
        pltpu.async_copy(data_v, out_hbm.at[idx_v], sem).wait()

    return sc_gather, sc_scatter


TT = 32


def _dense_body(logits_ref, datt_ref, enc_ref, dh_ref, de_ref, idsf_ref,
                lgg_ref, w1_ref, w2_ref, we_ref, bsum_ref,
                out_ref, val_ref, m_ref, we_s):
    hi = jax.lax.Precision.HIGHEST
    t = pl.program_id(1)

    @pl.when(t == 0)
    def _():
        ids = idsf_ref[0, 0, :]
        m_ref[...] = (ids[:, None] == ids[None, :]).astype(jnp.bfloat16)
        we_s[...] = lax.dot(enc_ref[0], we_ref[...], precision=hi)

    catt = jnp.mean(datt_ref[0], axis=0)
    z = (lax.dot(dh_ref[0], w1_ref[...], precision=hi)
         + lax.dot(de_ref[0], w2_ref[...], precision=hi)
         + lax.dot(catt, we_s[...], precision=hi)
         + bsum_ref[0, 0])
    p = jax.nn.sigmoid(z)

    l = logits_ref[0]
    m = jnp.max(l, axis=-1, keepdims=True)
    den = jnp.sum(jnp.exp(l - m), axis=-1, keepdims=True)
    a = (1.0 - p) / den
    out_ref[0] = (l - m) + jnp.log(a)

    cm = lax.dot(catt.astype(jnp.bfloat16), m_ref[...],
                 preferred_element_type=jnp.float32)
    val_ref[0] = jnp.log(a * jnp.exp(lgg_ref[0] - m) + p * cm + EPS)


def _dense_pass(logits, datt, enc, dh, de, idsf, lgg, w1, w2, we, bsum):
    grid = (B, T // TT)
    return pl.pallas_call(
        _dense_body,
        grid=grid,
        in_specs=[
            pl.BlockSpec((1, TT, V), lambda b, t: (b, t, 0)),
            pl.BlockSpec((1, H, TT, S), lambda b, t: (b, 0, t, 0)),
            pl.BlockSpec((1, S, D), lambda b, t: (b, 0, 0)),
            pl.BlockSpec((1, TT, D), lambda b, t: (b, t, 0)),
            pl.BlockSpec((1, TT, D), lambda b, t: (b, t, 0)),
            pl.BlockSpec((1, 1, S), lambda b, t: (b, 0, 0)),
            pl.BlockSpec((1, TT, S), lambda b, t: (b, t, 0)),
            pl.BlockSpec((D, 1), lambda b, t: (0, 0)),
            pl.BlockSpec((D, 1), lambda b, t: (0, 0)),
            pl.BlockSpec((D, 1), lambda b, t: (0, 0)),
            pl.BlockSpec(memory_space=pltpu.SMEM),
        ],
        out_specs=[
            pl.BlockSpec((1, TT, V), lambda b, t: (b, t, 0)),
            pl.BlockSpec((1, TT, S), lambda b, t: (b, t, 0)),
        ],
        out_shape=[
            jax.ShapeDtypeStruct((B, T, V), jnp.float32),
            jax.ShapeDtypeStruct((B, T, S), jnp.float32),
        ],
        scratch_shapes=[pltpu.VMEM((S, S), jnp.bfloat16),
                        pltpu.VMEM((S, 1), jnp.float32)],
    )(logits, datt, enc, dh, de, idsf, lgg, w1, w2, we, bsum)


def kernel(decoder_attention, decoder_last_hidden_state, enc_input_ids, logits,
           decoder_input_embeds, encoder_last_hidden_state,
           w_logits, b_logits, w_dec, b_dec, w_enc, b_enc, bias):
    ids32 = enc_input_ids.astype(jnp.int32)
    idsf = ids32.astype(jnp.float32).reshape(B, 1, S)
    bsum = (b_logits + b_dec + b_enc + bias).reshape(1, 1)

    sc_gather, sc_scatter = _sc_kernels()
    lg3 = sc_gather(logits.reshape(-1), ids32)
    lgg = lg3.reshape(B, T, S)

    out_pre, val = _dense_pass(
        logits, decoder_attention, encoder_last_hidden_state,
        decoder_last_hidden_state, decoder_input_embeds, idsf, lgg,
        w_logits, w_dec, w_enc, bsum)

    out_ref = jax.new_ref(out_pre.reshape(-1))
    sc_scatter(val.reshape(W, RW * S), ids32, out_ref)
    return out_ref[...].reshape(B, T, V)
